# Initial kernel scaffold; baseline (speedup 1.0000x reference)
import jax
import jax.numpy as jnp
from jax.experimental import pallas as pl


def kernel(edge_index, edge_type, norm, dynamic_emb, words_emb, rel_weight):
    raise NotImplementedError("write your pallas kernel here")


# speedup vs baseline: 41.3143x; 41.3143x over previous
import functools

import jax
import jax.numpy as jnp
from jax import lax
from jax.experimental import pallas as pl
from jax.experimental.pallas import tpu as pltpu
from jax.experimental.pallas import tpu_sc as plsc

N_ENTS = 9000
N = 10000
H = 128
R = 16
B = 8
S = 16
E = 320000
CH = 128
NW = 32
E_PAD = 323584
NCHUNK = E_PAD // (NW * CH)
N_ACC = 10240
ROWS_PER_SUB = N_ACC // 16
NEG_SLOPE = (1.0 / 8.0 + 1.0 / 3.0) / 2.0


def _table_body(h_ref, w_ref, o_ref):
    h = h_ref[...]
    for b in range(B):
        hb = h[:, b * S:(b + 1) * S]
        wb = w_ref[0, b]
        o_ref[0, :, b * S:(b + 1) * S] = jnp.dot(
            hb, wb, preferred_element_type=jnp.float32)


CHN = 2000


def _build_table(h, w4):
    out = pl.pallas_call(
        _table_body,
        grid=(N // CHN, R),
        in_specs=[
            pl.BlockSpec((CHN, H), lambda i, r: (i, 0)),
            pl.BlockSpec((1, B, S, S), lambda i, r: (r, 0, 0, 0)),
        ],
        out_specs=pl.BlockSpec((1, CHN, H), lambda i, r: (r, i, 0)),
        out_shape=jax.ShapeDtypeStruct((R, N, H), jnp.float32),
    )(h, w4)
    return out.reshape(R * N, H)


def _idx_body(et_ref, src_ref, o_ref):
    o_ref[...] = et_ref[...] * N + src_ref[...]


def _build_gidx(et2d, src2d):
    return pl.pallas_call(
        _idx_body,
        out_shape=jax.ShapeDtypeStruct(et2d.shape, jnp.int32),
    )(et2d, src2d)


_mesh = plsc.VectorSubcoreMesh(core_axis_name="c", subcore_axis_name="s")


@functools.partial(
    pl.kernel,
    mesh=_mesh,
    out_type=jax.ShapeDtypeStruct((2 * N_ACC, H), jnp.float32),
    scratch_types=[
        pltpu.VMEM((CH,), jnp.int32),
        pltpu.VMEM((CH,), jnp.int32),
        pltpu.VMEM((CH, H), jnp.float32),
        pltpu.VMEM_SHARED((N_ACC, H), jnp.float32),
        pltpu.SemaphoreType.DMA,
    ],
)
def _sc_gather_scatter(table, gidx2d, dst2d, zrows, out, gv, dv, rows, acc,
                       sem):
    c = lax.axis_index("c")
    s = lax.axis_index("s")
    wid = s * 2 + c
    pltpu.sync_copy(zrows, acc.at[pl.ds(s * ROWS_PER_SUB, ROWS_PER_SUB)])
    plsc.subcore_barrier()

    def body(j, carry):
        g = wid * NCHUNK + j
        pltpu.sync_copy(gidx2d.at[g], gv)
        pltpu.sync_copy(dst2d.at[g], dv)
        pltpu.async_copy(table.at[gv], rows, sem).wait()
        pltpu.sync_copy(rows, acc.at[dv], add=True)
        return carry

    lax.fori_loop(0, NCHUNK, body, 0)
    plsc.subcore_barrier()
    base = c * N_ACC + s * ROWS_PER_SUB
    pltpu.sync_copy(acc.at[pl.ds(s * ROWS_PER_SUB, ROWS_PER_SUB)],
                    out.at[pl.ds(base, ROWS_PER_SUB)])


CHF = 1000


def _final_body(p_ref, n_ref, o_ref):
    x = p_ref[0] + p_ref[1]
    x = x * n_ref[...]
    x = jnp.where(x >= 0, x, x * NEG_SLOPE)
    ss = jnp.sum(x * x, axis=1, keepdims=True)
    o_ref[...] = x / jnp.maximum(jnp.sqrt(ss), 1e-12)


def _finish(partials, norm):
    return pl.pallas_call(
        _final_body,
        grid=(N_ENTS // CHF,),
        in_specs=[
            pl.BlockSpec((2, CHF, H), lambda i: (0, i, 0)),
            pl.BlockSpec((CHF, 1), lambda i: (i, 0)),
        ],
        out_specs=pl.BlockSpec((CHF, H), lambda i: (i, 0)),
        out_shape=jax.ShapeDtypeStruct((N_ENTS, H), jnp.float32),
    )(partials, norm)


def kernel(edge_index, edge_type, norm, dynamic_emb, words_emb, rel_weight):
    h = jnp.concatenate([dynamic_emb, words_emb], axis=0)
    src = edge_index[0].astype(jnp.int32)
    dst = edge_index[1].astype(jnp.int32)
    et = edge_type.astype(jnp.int32)

    pad = E_PAD - E
    ar = jnp.arange(pad, dtype=jnp.int32)
    src_p = jnp.concatenate([src, ar % N])
    et_p = jnp.concatenate([et, jnp.zeros((pad,), jnp.int32)])
    dst_p = jnp.concatenate([dst, N + ar % (N_ACC - N)])

    w4 = rel_weight.reshape(R, B, S, S)
    table = _build_table(h, w4)
    gidx2d = _build_gidx(et_p.reshape(-1, CH), src_p.reshape(-1, CH))
    dst2d = dst_p.reshape(-1, CH)

    zrows = jnp.zeros((ROWS_PER_SUB, H), jnp.float32)
    flat = _sc_gather_scatter(table, gidx2d, dst2d, zrows)
    partials = flat.reshape(2, N_ACC, H)

    static_emb = _finish(partials, norm[:N_ENTS])
    return (static_emb, static_emb)


# speedup vs baseline: 105.7842x; 2.5605x over previous
import functools

import jax
import jax.numpy as jnp
from jax import lax
from jax.experimental import pallas as pl
from jax.experimental.pallas import tpu as pltpu
from jax.experimental.pallas import tpu_sc as plsc

N_ENTS = 9000
N = 10000
H = 128
R = 16
B = 8
S = 16
E = 320000
CH = 128
NW = 32
NCHUNK = 80
E_PAD = NW * NCHUNK * CH
NBUF = 2
NHALF = 2
CPH = NCHUNK // NHALF
N_ACC = 10240
ROWS_PER_SUB = N_ACC // 16
NEG_SLOPE = (1.0 / 8.0 + 1.0 / 3.0) / 2.0


def _table_body(h_ref, w_ref, o_ref, bd_ref):
    bd_ref[...] = jnp.zeros((H, H), jnp.float32)
    for b in range(B):
        bd_ref[b * S:(b + 1) * S, b * S:(b + 1) * S] = w_ref[0, b]
    o_ref[0] = jnp.dot(h_ref[...], bd_ref[...],
                       preferred_element_type=jnp.float32)


CHN = 2000


def _build_table(h, w4):
    out = pl.pallas_call(
        _table_body,
        grid=(N // CHN, R),
        in_specs=[
            pl.BlockSpec((CHN, H), lambda i, r: (i, 0)),
            pl.BlockSpec((1, B, S, S), lambda i, r: (r, 0, 0, 0)),
        ],
        out_specs=pl.BlockSpec((1, CHN, H), lambda i, r: (r, i, 0)),
        out_shape=jax.ShapeDtypeStruct((R, N, H), jnp.float32),
        scratch_shapes=[pltpu.VMEM((H, H), jnp.float32)],
    )(h, w4)
    return out.reshape(R * N, H)


def _idx_body(et_ref, src_ref, o_ref):
    o_ref[...] = et_ref[...] * N + src_ref[...]


def _build_gidx(et2d, src2d):
    return pl.pallas_call(
        _idx_body,
        out_shape=jax.ShapeDtypeStruct(et2d.shape, jnp.int32),
    )(et2d, src2d)


_mesh = plsc.VectorSubcoreMesh(core_axis_name="c", subcore_axis_name="s")


@functools.partial(
    pl.kernel,
    mesh=_mesh,
    out_type=jax.ShapeDtypeStruct((2 * N_ACC, H), jnp.float32),
    scratch_types=[
        pltpu.VMEM((CPH, CH), jnp.int32),
        pltpu.VMEM((CPH, CH), jnp.int32),
        pltpu.VMEM((NBUF, CH, H), jnp.float32),
        pltpu.VMEM_SHARED((N_ACC, H), jnp.float32),
    ]
    + [pltpu.SemaphoreType.DMA] * NBUF,
)
def _sc_gather_scatter(table, gidx2d, dst2d, zrows, out, gbuf, dbuf, rows,
                       acc, *sems):
    c = lax.axis_index("c")
    s = lax.axis_index("s")
    wid = s * 2 + c
    pltpu.sync_copy(zrows, acc.at[pl.ds(s * ROWS_PER_SUB, ROWS_PER_SUB)])
    plsc.subcore_barrier()

    ngrp = CPH // NBUF
    for half in range(NHALF):
        hbase = wid * NCHUNK + half * CPH
        pltpu.sync_copy(gidx2d.at[pl.ds(hbase, CPH)], gbuf)
        pltpu.sync_copy(dst2d.at[pl.ds(hbase, CPH)], dbuf)
        for q in range(NBUF):
            pltpu.async_copy(table.at[gbuf.at[q]], rows.at[q], sems[q])

        def body(p, carry):
            for q in range(NBUF):
                j = p * NBUF + q
                pltpu.make_async_copy(table.at[gbuf.at[j]], rows.at[q],
                                      sems[q]).wait()
                pltpu.sync_copy(rows.at[q], acc.at[dbuf.at[j]], add=True)

                @pl.when(p < ngrp - 1)
                def _():
                    pltpu.async_copy(table.at[gbuf.at[j + NBUF]],
                                     rows.at[q], sems[q])
            return carry

        lax.fori_loop(0, ngrp, body, 0)
    plsc.subcore_barrier()
    base = c * N_ACC + s * ROWS_PER_SUB
    pltpu.sync_copy(acc.at[pl.ds(s * ROWS_PER_SUB, ROWS_PER_SUB)],
                    out.at[pl.ds(base, ROWS_PER_SUB)])


CHF = 1000


def _final_body(p_ref, n_ref, o_ref):
    x = p_ref[0] + p_ref[1]
    x = x * n_ref[...]
    x = jnp.where(x >= 0, x, x * NEG_SLOPE)
    ss = jnp.sum(x * x, axis=1, keepdims=True)
    o_ref[...] = x / jnp.maximum(jnp.sqrt(ss), 1e-12)


def _finish(partials, norm):
    return pl.pallas_call(
        _final_body,
        grid=(N_ENTS // CHF,),
        in_specs=[
            pl.BlockSpec((2, CHF, H), lambda i: (0, i, 0)),
            pl.BlockSpec((CHF, 1), lambda i: (i, 0)),
        ],
        out_specs=pl.BlockSpec((CHF, H), lambda i: (i, 0)),
        out_shape=jax.ShapeDtypeStruct((N_ENTS, H), jnp.float32),
    )(partials, norm)


def kernel(edge_index, edge_type, norm, dynamic_emb, words_emb, rel_weight):
    h = jnp.concatenate([dynamic_emb, words_emb], axis=0)
    src = edge_index[0].astype(jnp.int32)
    dst = edge_index[1].astype(jnp.int32)
    et = edge_type.astype(jnp.int32)

    pad = E_PAD - E
    ar = jnp.arange(pad, dtype=jnp.int32)
    src_p = jnp.concatenate([src, ar % N])
    et_p = jnp.concatenate([et, jnp.zeros((pad,), jnp.int32)])
    dst_p = jnp.concatenate([dst, N + ar % (N_ACC - N)])

    w4 = rel_weight.reshape(R, B, S, S)
    table = _build_table(h, w4)
    gidx2d = _build_gidx(et_p.reshape(-1, CH), src_p.reshape(-1, CH))
    dst2d = dst_p.reshape(-1, CH)

    zrows = jnp.zeros((ROWS_PER_SUB, H), jnp.float32)
    flat = _sc_gather_scatter(table, gidx2d, dst2d, zrows)
    partials = flat.reshape(2, N_ACC, H)

    static_emb = _finish(partials, norm[:N_ENTS])
    return (static_emb, static_emb)


# speedup vs baseline: 106.4149x; 1.0060x over previous
import functools

import jax
import jax.numpy as jnp
from jax import lax
from jax.experimental import pallas as pl
from jax.experimental.pallas import tpu as pltpu
from jax.experimental.pallas import tpu_sc as plsc

N_ENTS = 9000
N = 10000
H = 128
R = 16
B = 8
S = 16
E = 320000
CH = 128
NW = 32
NCHUNK = 80
E_PAD = NW * NCHUNK * CH
NBUF = 2
NHALF = 2
CPH = NCHUNK // NHALF
N_ACC = 10240
ROWS_PER_SUB = N_ACC // 16
NEG_SLOPE = (1.0 / 8.0 + 1.0 / 3.0) / 2.0


CHN = 2000
NROW = E_PAD // CH
GCELLS = (N // CHN) * R
IROW = NROW // GCELLS


def _prep_body(h_ref, w_ref, et_ref, src_ref, o_ref, g_ref, bd_ref):
    bd_ref[...] = jnp.zeros((H, H), jnp.float32)
    for b in range(B):
        bd_ref[b * S:(b + 1) * S, b * S:(b + 1) * S] = w_ref[0, b]
    o_ref[0] = jnp.dot(h_ref[...], bd_ref[...],
                       preferred_element_type=jnp.float32)
    g_ref[...] = et_ref[...] * N + src_ref[...]


def _build_table_gidx(h, w4, et2d, src2d):
    table, gidx2d = pl.pallas_call(
        _prep_body,
        grid=(N // CHN, R),
        in_specs=[
            pl.BlockSpec((CHN, H), lambda i, r: (i, 0)),
            pl.BlockSpec((1, B, S, S), lambda i, r: (r, 0, 0, 0)),
            pl.BlockSpec((IROW, CH), lambda i, r: (i * R + r, 0)),
            pl.BlockSpec((IROW, CH), lambda i, r: (i * R + r, 0)),
        ],
        out_specs=[
            pl.BlockSpec((1, CHN, H), lambda i, r: (r, i, 0)),
            pl.BlockSpec((IROW, CH), lambda i, r: (i * R + r, 0)),
        ],
        out_shape=[
            jax.ShapeDtypeStruct((R, N, H), jnp.float32),
            jax.ShapeDtypeStruct((NROW, CH), jnp.int32),
        ],
        scratch_shapes=[pltpu.VMEM((H, H), jnp.float32)],
    )(h, w4, et2d, src2d)
    return table.reshape(R * N, H), gidx2d


_mesh = plsc.VectorSubcoreMesh(core_axis_name="c", subcore_axis_name="s")


@functools.partial(
    pl.kernel,
    mesh=_mesh,
    out_type=jax.ShapeDtypeStruct((2 * N_ACC, H), jnp.float32),
    scratch_types=[
        pltpu.VMEM((CPH, CH), jnp.int32),
        pltpu.VMEM((CPH, CH), jnp.int32),
        pltpu.VMEM((NBUF, CH, H), jnp.float32),
        pltpu.VMEM_SHARED((N_ACC, H), jnp.float32),
    ]
    + [pltpu.SemaphoreType.DMA] * NBUF,
)
def _sc_gather_scatter(table, gidx2d, dst2d, zrows, out, gbuf, dbuf, rows,
                       acc, *sems):
    c = lax.axis_index("c")
    s = lax.axis_index("s")
    wid = s * 2 + c
    pltpu.sync_copy(zrows, acc.at[pl.ds(s * ROWS_PER_SUB, ROWS_PER_SUB)])
    plsc.subcore_barrier()

    ngrp = CPH // NBUF
    for half in range(NHALF):
        hbase = wid * NCHUNK + half * CPH
        pltpu.sync_copy(gidx2d.at[pl.ds(hbase, CPH)], gbuf)
        pltpu.sync_copy(dst2d.at[pl.ds(hbase, CPH)], dbuf)
        for q in range(NBUF):
            pltpu.async_copy(table.at[gbuf.at[q]], rows.at[q], sems[q])

        def body(p, carry):
            for q in range(NBUF):
                j = p * NBUF + q
                pltpu.make_async_copy(table.at[gbuf.at[j]], rows.at[q],
                                      sems[q]).wait()
                pltpu.sync_copy(rows.at[q], acc.at[dbuf.at[j]], add=True)

                @pl.when(p < ngrp - 1)
                def _():
                    pltpu.async_copy(table.at[gbuf.at[j + NBUF]],
                                     rows.at[q], sems[q])
            return carry

        lax.fori_loop(0, ngrp, body, 0)
    plsc.subcore_barrier()
    base = c * N_ACC + s * ROWS_PER_SUB
    pltpu.sync_copy(acc.at[pl.ds(s * ROWS_PER_SUB, ROWS_PER_SUB)],
                    out.at[pl.ds(base, ROWS_PER_SUB)])


CHF = 1000


def _final_body(p_ref, n_ref, o_ref):
    x = p_ref[0] + p_ref[1]
    x = x * n_ref[...]
    x = jnp.where(x >= 0, x, x * NEG_SLOPE)
    ss = jnp.sum(x * x, axis=1, keepdims=True)
    o_ref[...] = x / jnp.maximum(jnp.sqrt(ss), 1e-12)


def _finish(partials, norm):
    return pl.pallas_call(
        _final_body,
        grid=(N_ENTS // CHF,),
        in_specs=[
            pl.BlockSpec((2, CHF, H), lambda i: (0, i, 0)),
            pl.BlockSpec((CHF, 1), lambda i: (i, 0)),
        ],
        out_specs=pl.BlockSpec((CHF, H), lambda i: (i, 0)),
        out_shape=jax.ShapeDtypeStruct((N_ENTS, H), jnp.float32),
    )(partials, norm)


def kernel(edge_index, edge_type, norm, dynamic_emb, words_emb, rel_weight):
    h = jnp.concatenate([dynamic_emb, words_emb], axis=0)
    src = edge_index[0].astype(jnp.int32)
    dst = edge_index[1].astype(jnp.int32)
    et = edge_type.astype(jnp.int32)

    pad = E_PAD - E
    ar = jnp.arange(pad, dtype=jnp.int32)
    src_p = jnp.concatenate([src, ar % N])
    et_p = jnp.concatenate([et, jnp.zeros((pad,), jnp.int32)])
    dst_p = jnp.concatenate([dst, N + ar % (N_ACC - N)])

    w4 = rel_weight.reshape(R, B, S, S)
    table, gidx2d = _build_table_gidx(
        h, w4, et_p.reshape(-1, CH), src_p.reshape(-1, CH))
    dst2d = dst_p.reshape(-1, CH)

    zrows = jnp.zeros((ROWS_PER_SUB, H), jnp.float32)
    flat = _sc_gather_scatter(table, gidx2d, dst2d, zrows)
    partials = flat.reshape(2, N_ACC, H)

    static_emb = _finish(partials, norm[:N_ENTS])
    return (static_emb, static_emb)


# speedup vs baseline: 126.7529x; 1.1911x over previous
import functools

import jax
import jax.numpy as jnp
from jax import lax
from jax.experimental import pallas as pl
from jax.experimental.pallas import tpu as pltpu
from jax.experimental.pallas import tpu_sc as plsc

N_ENTS = 9000
N = 10000
H = 128
R = 16
B = 8
S = 16
E = 320000
CH = 128
NW = 32
NCHUNK = 80
E_PAD = NW * NCHUNK * CH
NBUF = 2
NHALF = 2
CPH = NCHUNK // NHALF
N_ACC = 10240
ROWS_PER_SUB = N_ACC // 16
NEG_SLOPE = (1.0 / 8.0 + 1.0 / 3.0) / 2.0


NROW = E_PAD // CH
IROW = NROW // R


def _prep_body(h_ref, w_ref, et_ref, src_ref, o_ref, g_ref, bd_ref):
    bd_ref[...] = jnp.zeros((H, H), jnp.float32)
    for b in range(B):
        bd_ref[b * S:(b + 1) * S, b * S:(b + 1) * S] = w_ref[0, b]
    o_ref[0] = jnp.dot(h_ref[...], bd_ref[...],
                       preferred_element_type=jnp.float32)
    g_ref[...] = et_ref[...] * N + src_ref[...]


def _build_table_gidx(h, w4, et2d, src2d):
    table, gidx2d = pl.pallas_call(
        _prep_body,
        grid=(R,),
        in_specs=[
            pl.BlockSpec((N, H), lambda r: (0, 0)),
            pl.BlockSpec((1, B, S, S), lambda r: (r, 0, 0, 0)),
            pl.BlockSpec((IROW, CH), lambda r: (r, 0)),
            pl.BlockSpec((IROW, CH), lambda r: (r, 0)),
        ],
        out_specs=[
            pl.BlockSpec((1, N, H), lambda r: (r, 0, 0)),
            pl.BlockSpec((IROW, CH), lambda r: (r, 0)),
        ],
        out_shape=[
            jax.ShapeDtypeStruct((R, N, H), jnp.float32),
            jax.ShapeDtypeStruct((NROW, CH), jnp.int32),
        ],
        scratch_shapes=[pltpu.VMEM((H, H), jnp.float32)],
    )(h, w4, et2d, src2d)
    return table.reshape(R * N, H), gidx2d


_mesh = plsc.VectorSubcoreMesh(core_axis_name="c", subcore_axis_name="s")


@functools.partial(
    pl.kernel,
    mesh=_mesh,
    out_type=jax.ShapeDtypeStruct((2 * N_ACC, H), jnp.float32),
    scratch_types=[
        pltpu.VMEM((CPH, CH), jnp.int32),
        pltpu.VMEM((CPH, CH), jnp.int32),
        pltpu.VMEM((NBUF, CH, H), jnp.float32),
        pltpu.VMEM_SHARED((N_ACC, H), jnp.float32),
    ]
    + [pltpu.SemaphoreType.DMA] * NBUF,
)
def _sc_gather_scatter(table, gidx2d, dst2d, zrows, out, gbuf, dbuf, rows,
                       acc, *sems):
    c = lax.axis_index("c")
    s = lax.axis_index("s")
    wid = s * 2 + c
    pltpu.sync_copy(zrows, acc.at[pl.ds(s * ROWS_PER_SUB, ROWS_PER_SUB)])
    plsc.subcore_barrier()

    ngrp = CPH // NBUF
    for half in range(NHALF):
        hbase = wid * NCHUNK + half * CPH
        pltpu.sync_copy(gidx2d.at[pl.ds(hbase, CPH)], gbuf)
        pltpu.sync_copy(dst2d.at[pl.ds(hbase, CPH)], dbuf)
        for q in range(NBUF):
            pltpu.async_copy(table.at[gbuf.at[q]], rows.at[q], sems[q])

        def body(p, carry):
            for q in range(NBUF):
                j = p * NBUF + q
                pltpu.make_async_copy(table.at[gbuf.at[j]], rows.at[q],
                                      sems[q]).wait()
                pltpu.sync_copy(rows.at[q], acc.at[dbuf.at[j]], add=True)

                @pl.when(p < ngrp - 1)
                def _():
                    pltpu.async_copy(table.at[gbuf.at[j + NBUF]],
                                     rows.at[q], sems[q])
            return carry

        lax.fori_loop(0, ngrp, body, 0)
    plsc.subcore_barrier()
    base = c * N_ACC + s * ROWS_PER_SUB
    pltpu.sync_copy(acc.at[pl.ds(s * ROWS_PER_SUB, ROWS_PER_SUB)],
                    out.at[pl.ds(base, ROWS_PER_SUB)])


CHF = 1000


def _final_body(p_ref, n_ref, o_ref):
    x = p_ref[0] + p_ref[1]
    x = x * n_ref[...]
    x = jnp.where(x >= 0, x, x * NEG_SLOPE)
    ss = jnp.sum(x * x, axis=1, keepdims=True)
    o_ref[...] = x / jnp.maximum(jnp.sqrt(ss), 1e-12)


def _finish(partials, norm):
    return pl.pallas_call(
        _final_body,
        grid=(N_ENTS // CHF,),
        in_specs=[
            pl.BlockSpec((2, CHF, H), lambda i: (0, i, 0)),
            pl.BlockSpec((CHF, 1), lambda i: (i, 0)),
        ],
        out_specs=pl.BlockSpec((CHF, H), lambda i: (i, 0)),
        out_shape=jax.ShapeDtypeStruct((N_ENTS, H), jnp.float32),
    )(partials, norm)


def kernel(edge_index, edge_type, norm, dynamic_emb, words_emb, rel_weight):
    h = jnp.concatenate([dynamic_emb, words_emb], axis=0)
    src = edge_index[0].astype(jnp.int32)
    dst = edge_index[1].astype(jnp.int32)
    et = edge_type.astype(jnp.int32)

    pad = E_PAD - E
    ar = jnp.arange(pad, dtype=jnp.int32)
    src_p = jnp.concatenate([src, ar % N])
    et_p = jnp.concatenate([et, jnp.zeros((pad,), jnp.int32)])
    dst_p = jnp.concatenate([dst, N + ar % (N_ACC - N)])

    w4 = rel_weight.reshape(R, B, S, S)
    table, gidx2d = _build_table_gidx(
        h, w4, et_p.reshape(-1, CH), src_p.reshape(-1, CH))
    dst2d = dst_p.reshape(-1, CH)

    zrows = jnp.zeros((ROWS_PER_SUB, H), jnp.float32)
    flat = _sc_gather_scatter(table, gidx2d, dst2d, zrows)
    partials = flat.reshape(2, N_ACC, H)

    static_emb = _finish(partials, norm[:N_ENTS])
    return (static_emb, static_emb)
